# baseline (speedup 1.0000x reference)
import jax
import jax.numpy as jnp
from jax import lax
from jax.experimental import pallas as pl
from jax.experimental.pallas import tpu as pltpu
from jax.experimental.pallas import tpu_sc as plsc

N_CLASSES = 1000
BS = 16384
NBINS = 1024
L = 16
NW = 32
OCHUNK = BS // NW
NCHUNK = 4
CH = BS // NCHUNK


def _body(loss_hbm, tb_hbm, ep_hbm, out_hbm,
          tb_v, hist, tb_o, loss_v, out_v, ep_v,
          semt, semo):
    s = lax.axis_index("s")
    c = lax.axis_index("c")
    wid = s * 2 + c

    ones = jnp.full((L,), 1.0, jnp.float32)
    zeros = jnp.zeros((L,), jnp.float32)

    ep_v[pl.ds(0, L)] = jnp.zeros((L,), jnp.int32)
    d_tb = [pltpu.async_copy(tb_hbm.at[pl.ds(i * CH, CH)],
                             tb_v.at[pl.ds(i * CH, CH)], semt[i])
            for i in range(NCHUNK)]
    d_to = pltpu.async_copy(tb_hbm.at[pl.ds(wid * OCHUNK, OCHUNK)], tb_o, semo)
    d_ls = pltpu.async_copy(loss_hbm.at[pl.ds(wid * OCHUNK, OCHUNK)], loss_v, semo)
    d_ep = pltpu.async_copy(ep_hbm, ep_v.at[pl.ds(0, 1)], semo)

    for k in range(NBINS // L):
        hist[pl.ds(k * L, L)] = zeros

    for i in range(NCHUNK):
        d_tb[i].wait()

        def _scat(k, carry):
            idx = tb_v[pl.ds(i * CH + k * L, L)]
            plsc.addupdate_scatter(hist, [idx], ones)
            return carry
        lax.fori_loop(0, CH // L, _scat, 0, unroll=16)

    wsum = jnp.zeros((L,), jnp.float32)
    inv_total = jnp.float32(1.0 / BS)
    for k in range(NBINS // L):
        cv = hist[pl.ds(k * L, L)]
        pos = cv > 0.0
        prob = jnp.where(pos, cv * inv_total, ones)
        rv = jnp.where(pos, 1.0 / prob, zeros)
        hist[pl.ds(k * L, L)] = rv
        wsum = wsum + cv * rv
    mean_v = jnp.full((L,), jnp.sum(wsum), jnp.float32) * inv_total
    inv_mean = ones / mean_v

    d_to.wait()
    d_ls.wait()
    d_ep.wait()
    ep_s = jnp.sum(ep_v[pl.ds(0, L)])
    epoch0 = jnp.full((L,), ep_s, jnp.int32) == 0
    alt = jnp.full((L,), 1.0 / N_CLASSES, jnp.float32)
    for k in range(OCHUNK // L):
        idx = tb_o[pl.ds(k * L, L)]
        rv = plsc.load_gather(hist, [idx])
        scale = jnp.where(epoch0, rv * inv_mean, alt)
        out_v[pl.ds(k * L, L)] = loss_v[pl.ds(k * L, L)] * scale
    pltpu.sync_copy(out_v, out_hbm.at[pl.ds(wid * OCHUNK, OCHUNK)])


_sc_call = pl.kernel(
    _body,
    out_type=jax.ShapeDtypeStruct((BS,), jnp.float32),
    mesh=plsc.VectorSubcoreMesh(core_axis_name="c", subcore_axis_name="s"),
    compiler_params=pltpu.CompilerParams(needs_layout_passes=False),
    scratch_types=[
        pltpu.VMEM((BS,), jnp.int32),
        pltpu.VMEM((NBINS,), jnp.float32),
        pltpu.VMEM((OCHUNK,), jnp.int32),
        pltpu.VMEM((OCHUNK,), jnp.float32),
        pltpu.VMEM((OCHUNK,), jnp.float32),
        pltpu.VMEM((L,), jnp.int32),
        [pltpu.SemaphoreType.DMA] * NCHUNK,
        pltpu.SemaphoreType.DMA,
    ],
)


def kernel(loss, Tb, i_current_epoch):
    ep = jnp.reshape(jnp.asarray(i_current_epoch, jnp.int32), (1,))
    return _sc_call(loss, Tb, ep)


# speedup vs baseline: 1.3458x; 1.3458x over previous
import jax
import jax.numpy as jnp
from jax import lax
from jax.experimental import pallas as pl
from jax.experimental.pallas import tpu as pltpu
from jax.experimental.pallas import tpu_sc as plsc

N_CLASSES = 1000
BS = 16384
NBINS = 1024
L = 16
NSUB = 16
NW = 32
HCHUNK = BS // NSUB
OCHUNK = BS // NW


def _body(loss_hbm, tb_hbm, ep_hbm, out_hbm,
          tb_v, hist, colblk, cnt64, cnt_v, tb_o, loss_v, out_v, ep_v,
          psum, pbuf, shared, counts_sh, parts_sh, sem1, sem2, sem3):
    s = lax.axis_index("s")
    c = lax.axis_index("c")
    wid = s * 2 + c

    ones = jnp.full((L,), 1.0, jnp.float32)
    zeros = jnp.zeros((L,), jnp.float32)

    ep_v[pl.ds(0, L)] = jnp.zeros((L,), jnp.int32)
    d_tb = pltpu.async_copy(tb_hbm.at[pl.ds(s * HCHUNK, HCHUNK)], tb_v, sem1)
    d_to = pltpu.async_copy(tb_hbm.at[pl.ds(wid * OCHUNK, OCHUNK)], tb_o, sem2)
    d_ls = pltpu.async_copy(loss_hbm.at[pl.ds(wid * OCHUNK, OCHUNK)], loss_v, sem2)
    d_ep = pltpu.async_copy(ep_hbm, ep_v.at[pl.ds(0, 1)], sem2)

    for k in range(NBINS // L):
        hist[pl.ds(k * L, L)] = zeros
    d_tb.wait()
    for k in range(HCHUNK // L):
        idx = tb_v[pl.ds(k * L, L)]
        plsc.addupdate_scatter(hist, [idx], ones)
    pltpu.sync_copy(hist, shared.at[pl.ds(s * NBINS, NBINS)])
    plsc.subcore_barrier()

    fan = [pltpu.async_copy(shared.at[pl.ds(r * NBINS + s * 64, 64)],
                            colblk.at[pl.ds(r * 64, 64)], sem3)
           for r in range(NSUB)]
    for d in fan:
        d.wait()
    acc = [jnp.zeros((L,), jnp.float32) for _ in range(4)]
    for r in range(NSUB):
        for v in range(4):
            acc[v] = acc[v] + colblk[pl.ds(r * 64 + v * L, L)]
    inv_total = jnp.float32(1.0 / BS)
    wpart = jnp.zeros((L,), jnp.float32)
    for v in range(4):
        cv = acc[v]
        pos = cv > 0.0
        prob = jnp.where(pos, cv * inv_total, ones)
        rv = jnp.where(pos, 1.0 / prob, zeros)
        cnt64[pl.ds(v * L, L)] = rv
        wpart = wpart + cv * rv
    psum[pl.ds(0, L)] = wpart
    pltpu.sync_copy(cnt64, counts_sh.at[pl.ds(s * 64, 64)])
    pltpu.sync_copy(psum, parts_sh.at[pl.ds(s * L, L)])
    plsc.subcore_barrier()

    d_rec = pltpu.async_copy(counts_sh, cnt_v, sem1)
    d_ps = pltpu.async_copy(parts_sh, pbuf, sem3)
    d_rec.wait()
    d_ps.wait()
    wsum = jnp.zeros((L,), jnp.float32)
    for r in range(NSUB):
        wsum = wsum + pbuf[pl.ds(r * L, L)]
    mean_v = jnp.full((L,), jnp.sum(wsum), jnp.float32) * inv_total
    inv_mean = ones / mean_v

    d_to.wait()
    d_ls.wait()
    d_ep.wait()
    ep_s = jnp.sum(ep_v[pl.ds(0, L)])
    epoch0 = jnp.full((L,), ep_s, jnp.int32) == 0
    alt = jnp.full((L,), 1.0 / N_CLASSES, jnp.float32)
    for k in range(OCHUNK // L):
        idx = tb_o[pl.ds(k * L, L)]
        rv = plsc.load_gather(cnt_v, [idx])
        scale = jnp.where(epoch0, rv * inv_mean, alt)
        out_v[pl.ds(k * L, L)] = loss_v[pl.ds(k * L, L)] * scale
    pltpu.sync_copy(out_v, out_hbm.at[pl.ds(wid * OCHUNK, OCHUNK)])


_sc_call = pl.kernel(
    _body,
    out_type=jax.ShapeDtypeStruct((BS,), jnp.float32),
    mesh=plsc.VectorSubcoreMesh(core_axis_name="c", subcore_axis_name="s"),
    compiler_params=pltpu.CompilerParams(needs_layout_passes=False),
    scratch_types=[
        pltpu.VMEM((HCHUNK,), jnp.int32),
        pltpu.VMEM((NBINS,), jnp.float32),
        pltpu.VMEM((NSUB * 64,), jnp.float32),
        pltpu.VMEM((64,), jnp.float32),
        pltpu.VMEM((NBINS,), jnp.float32),
        pltpu.VMEM((OCHUNK,), jnp.int32),
        pltpu.VMEM((OCHUNK,), jnp.float32),
        pltpu.VMEM((OCHUNK,), jnp.float32),
        pltpu.VMEM((L,), jnp.int32),
        pltpu.VMEM((L,), jnp.float32),
        pltpu.VMEM((NSUB * L,), jnp.float32),
        pltpu.VMEM_SHARED((NSUB * NBINS,), jnp.float32),
        pltpu.VMEM_SHARED((NBINS,), jnp.float32),
        pltpu.VMEM_SHARED((NSUB * L,), jnp.float32),
        pltpu.SemaphoreType.DMA,
        pltpu.SemaphoreType.DMA,
        pltpu.SemaphoreType.DMA,
    ],
)


def kernel(loss, Tb, i_current_epoch):
    ep = jnp.reshape(jnp.asarray(i_current_epoch, jnp.int32), (1,))
    return _sc_call(loss, Tb, ep)
